# Initial kernel scaffold; baseline (speedup 1.0000x reference)
import jax
import jax.numpy as jnp
from jax.experimental import pallas as pl


def kernel(V, H):
    raise NotImplementedError("write your pallas kernel here")


# speedup vs baseline: 1.8481x; 1.8481x over previous
import functools

import jax
import jax.numpy as jnp
from jax import lax
from jax.experimental import pallas as pl
from jax.experimental.pallas import tpu as pltpu
from jax.experimental.pallas import tpu_sc as plsc

_B, _C, _HD, _WD = 8, 96, 224, 224
_HW = _HD * _WD
_NW = 32
_WPB = _NW // _B
_CPW = _C // _WPB
_CHUNK = 12544
_NCHUNK = _HW // _CHUNK


def _sc_gather(v_flat, h):
    mesh = plsc.VectorSubcoreMesh(core_axis_name="c", subcore_axis_name="s")

    @functools.partial(
        pl.kernel,
        mesh=mesh,
        out_type=jax.ShapeDtypeStruct((_B * _C, _HW), jnp.float32),
        scratch_types=[
            pltpu.VMEM((_HW,), jnp.int32),
            pltpu.VMEM((_HW,), jnp.float32),
            pltpu.VMEM((_CHUNK,), jnp.float32),
        ],
        compiler_params=pltpu.CompilerParams(needs_layout_passes=False),
    )
    def k(v_hbm, h_hbm, out_hbm, idx_v, row_v, out_v):
        cid = lax.axis_index("c")
        sid = lax.axis_index("s")
        wid = sid * 2 + cid
        b = wid // _WPB
        part = wid % _WPB
        pltpu.sync_copy(h_hbm.at[b], idx_v)

        def chan_body(j, carry):
            r = b * _C + part * _CPW + j
            pltpu.sync_copy(v_hbm.at[r], row_v)

            def chunk_body(kk, carry2):
                def gather_body(i, carry3):
                    src = idx_v[pl.ds(kk * _CHUNK + i * 16, 16)]
                    out_v[pl.ds(i * 16, 16)] = plsc.load_gather(row_v, [src])
                    return carry3

                lax.fori_loop(0, _CHUNK // 16, gather_body, 0, unroll=8)
                pltpu.sync_copy(out_v, out_hbm.at[r, pl.ds(kk * _CHUNK, _CHUNK)])
                return carry2

            lax.fori_loop(0, _NCHUNK, chunk_body, 0)
            return carry

        lax.fori_loop(0, _CPW, chan_body, 0)

    return k(v_flat, h)


def kernel(V, H):
    b, c, hd, wd = V.shape
    v_flat = V.reshape(b * c, hd * wd)
    out = _sc_gather(v_flat, H)
    return out.reshape(b, c, hd, wd)


# speedup vs baseline: 3.6627x; 1.9819x over previous
import functools

import jax
import jax.numpy as jnp
from jax import lax
from jax.experimental import pallas as pl
from jax.experimental.pallas import tpu as pltpu
from jax.experimental.pallas import tpu_sc as plsc

_B, _C, _HD, _WD = 8, 96, 224, 224
_HW = _HD * _WD
_NW = 32
_WPB = _NW // _B
_CPW = _C // _WPB
_CHUNK = 12544
_NCHUNK = _HW // _CHUNK


def _sc_gather(v_flat, h):
    mesh = plsc.VectorSubcoreMesh(core_axis_name="c", subcore_axis_name="s")

    @functools.partial(
        pl.kernel,
        mesh=mesh,
        out_type=jax.ShapeDtypeStruct((_B * _C, _HW), jnp.float32),
        scratch_types=[
            pltpu.VMEM((_HW,), jnp.int32),
            pltpu.VMEM((_HW,), jnp.float32),
            pltpu.VMEM((_CHUNK,), jnp.float32),
        ],
        compiler_params=pltpu.CompilerParams(needs_layout_passes=False),
    )
    def k(v_hbm, h_hbm, out_hbm, idx_v, row_v, out_v):
        cid = lax.axis_index("c")
        sid = lax.axis_index("s")
        wid = sid * 2 + cid
        b = wid // _WPB
        part = wid % _WPB
        pltpu.sync_copy(h_hbm.at[b], idx_v)

        def chan_body(j, carry):
            r = b * _C + part * _CPW + j
            pltpu.sync_copy(v_hbm.at[r], row_v)

            def chunk_body(kk, carry2):
                @plsc.parallel_loop(0, _CHUNK // 16, unroll=8)
                def _(i):
                    src = idx_v[pl.ds(kk * _CHUNK + i * 16, 16)]
                    out_v[pl.ds(i * 16, 16)] = plsc.load_gather(row_v, [src])
                pltpu.sync_copy(out_v, out_hbm.at[r, pl.ds(kk * _CHUNK, _CHUNK)])
                return carry2

            lax.fori_loop(0, _NCHUNK, chunk_body, 0)
            return carry

        lax.fori_loop(0, _CPW, chan_body, 0)

    return k(v_flat, h)


def kernel(V, H):
    b, c, hd, wd = V.shape
    v_flat = V.reshape(b * c, hd * wd)
    out = _sc_gather(v_flat, H)
    return out.reshape(b, c, hd, wd)


# speedup vs baseline: 3.9703x; 1.0840x over previous
import functools

import jax
import jax.numpy as jnp
from jax import lax
from jax.experimental import pallas as pl
from jax.experimental.pallas import tpu as pltpu
from jax.experimental.pallas import tpu_sc as plsc

_B, _C, _HD, _WD = 8, 96, 224, 224
_HW = _HD * _WD
_NW = 32
_WPB = _NW // _B
_CPW = _C // _WPB
_NOBUF = 4
_CHUNK = 6272
_NCHUNK = _HW // _CHUNK
_RSPLIT = 8
_RSUB = _HW // _RSPLIT
_ISPLIT = 4
_ISUB = _HW // _ISPLIT


def _sc_gather(v_flat, h):
    mesh = plsc.VectorSubcoreMesh(core_axis_name="c", subcore_axis_name="s")

    @functools.partial(
        pl.kernel,
        mesh=mesh,
        out_type=jax.ShapeDtypeStruct((_B * _C, _HW), jnp.float32),
        scratch_types=[
            pltpu.VMEM((_HW,), jnp.int32),
            pltpu.VMEM((_HW,), jnp.float32),
            [pltpu.VMEM((_CHUNK,), jnp.float32)] * _NOBUF,
            [pltpu.SemaphoreType.DMA] * _NOBUF,
            pltpu.SemaphoreType.DMA,
        ],
        compiler_params=pltpu.CompilerParams(needs_layout_passes=False),
    )
    def k(v_hbm, h_hbm, out_hbm, idx_v, row_v, outs, osems, rsem):
        cid = lax.axis_index("c")
        sid = lax.axis_index("s")
        wid = sid * 2 + cid
        b = wid // _WPB
        part = wid % _WPB

        icps = [
            pltpu.async_copy(
                h_hbm.at[b, pl.ds(t * _ISUB, _ISUB)],
                idx_v.at[pl.ds(t * _ISUB, _ISUB)],
                rsem,
            )
            for t in range(_ISPLIT)
        ]
        for cp in icps:
            cp.wait()

        def chan_body(j, carry):
            r = b * _C + part * _CPW + j
            rcps = [
                pltpu.async_copy(
                    v_hbm.at[r, pl.ds(t * _RSUB, _RSUB)],
                    row_v.at[pl.ds(t * _RSUB, _RSUB)],
                    rsem,
                )
                for t in range(_RSPLIT)
            ]
            for cp in rcps:
                cp.wait()

            ocps = [None] * _NCHUNK
            for kk in range(_NCHUNK):
                ov = outs[kk % _NOBUF]
                if kk >= _NOBUF:
                    ocps[kk - _NOBUF].wait()
                else:
                    @pl.when(j > 0)
                    def _():
                        pltpu.make_async_copy(
                            ov,
                            out_hbm.at[r, pl.ds(kk * _CHUNK, _CHUNK)],
                            osems[kk % _NOBUF],
                        ).wait()

                @plsc.parallel_loop(0, _CHUNK // 16, unroll=8)
                def _(i):
                    src = idx_v[pl.ds(kk * _CHUNK + i * 16, 16)]
                    ov[pl.ds(i * 16, 16)] = plsc.load_gather(row_v, [src])

                ocps[kk] = pltpu.async_copy(
                    ov, out_hbm.at[r, pl.ds(kk * _CHUNK, _CHUNK)], osems[kk % _NOBUF]
                )
            return carry

        lax.fori_loop(0, _CPW, chan_body, 0)

        r_last = b * _C + part * _CPW + (_CPW - 1)
        for kk in range(_NCHUNK - _NOBUF, _NCHUNK):
            pltpu.make_async_copy(
                outs[kk % _NOBUF],
                out_hbm.at[r_last, pl.ds(kk * _CHUNK, _CHUNK)],
                osems[kk % _NOBUF],
            ).wait()

    return k(v_flat, h)


def kernel(V, H):
    b, c, hd, wd = V.shape
    v_flat = V.reshape(b * c, hd * wd)
    out = _sc_gather(v_flat, H)
    return out.reshape(b, c, hd, wd)


# speedup vs baseline: 3.9710x; 1.0002x over previous
import functools

import jax
import jax.numpy as jnp
from jax import lax
from jax.experimental import pallas as pl
from jax.experimental.pallas import tpu as pltpu
from jax.experimental.pallas import tpu_sc as plsc

_B, _C, _HD, _WD = 8, 96, 224, 224
_HW = _HD * _WD
_NW = 32
_WPB = _NW // _B
_CPW = _C // _WPB
_NOBUF = 4
_CHUNK = 6272
_NCHUNK = _HW // _CHUNK
_RSPLIT = 8
_RSUB = _HW // _RSPLIT
_ISPLIT = 4
_ISUB = _HW // _ISPLIT


def _sc_gather(v_flat, h):
    mesh = plsc.VectorSubcoreMesh(core_axis_name="c", subcore_axis_name="s")

    @functools.partial(
        pl.kernel,
        mesh=mesh,
        out_type=jax.ShapeDtypeStruct((_B * _C, _HW), jnp.float32),
        scratch_types=[
            pltpu.VMEM((_HW,), jnp.int32),
            pltpu.VMEM((_HW,), jnp.float32),
            [pltpu.VMEM((_CHUNK,), jnp.float32)] * _NOBUF,
            [pltpu.SemaphoreType.DMA] * _NOBUF,
            pltpu.SemaphoreType.DMA,
        ],
        compiler_params=pltpu.CompilerParams(needs_layout_passes=False, use_tc_tiling_on_sc=True),
    )
    def k(v_hbm, h_hbm, out_hbm, idx_v, row_v, outs, osems, rsem):
        cid = lax.axis_index("c")
        sid = lax.axis_index("s")
        wid = sid * 2 + cid
        b = wid // _WPB
        part = wid % _WPB

        icps = [
            pltpu.async_copy(
                h_hbm.at[b, pl.ds(t * _ISUB, _ISUB)],
                idx_v.at[pl.ds(t * _ISUB, _ISUB)],
                rsem,
            )
            for t in range(_ISPLIT)
        ]
        for cp in icps:
            cp.wait()

        def chan_body(j, carry):
            r = b * _C + part * _CPW + j
            rcps = [
                pltpu.async_copy(
                    v_hbm.at[r, pl.ds(t * _RSUB, _RSUB)],
                    row_v.at[pl.ds(t * _RSUB, _RSUB)],
                    rsem,
                )
                for t in range(_RSPLIT)
            ]
            for cp in rcps:
                cp.wait()

            ocps = [None] * _NCHUNK
            for kk in range(_NCHUNK):
                ov = outs[kk % _NOBUF]
                if kk >= _NOBUF:
                    ocps[kk - _NOBUF].wait()
                else:
                    @pl.when(j > 0)
                    def _():
                        pltpu.make_async_copy(
                            ov,
                            out_hbm.at[r, pl.ds(kk * _CHUNK, _CHUNK)],
                            osems[kk % _NOBUF],
                        ).wait()

                @plsc.parallel_loop(0, _CHUNK // 16, unroll=8)
                def _(i):
                    src = idx_v[pl.ds(kk * _CHUNK + i * 16, 16)]
                    ov[pl.ds(i * 16, 16)] = plsc.load_gather(row_v, [src])

                ocps[kk] = pltpu.async_copy(
                    ov, out_hbm.at[r, pl.ds(kk * _CHUNK, _CHUNK)], osems[kk % _NOBUF]
                )
            return carry

        lax.fori_loop(0, _CPW, chan_body, 0)

        r_last = b * _C + part * _CPW + (_CPW - 1)
        for kk in range(_NCHUNK - _NOBUF, _NCHUNK):
            pltpu.make_async_copy(
                outs[kk % _NOBUF],
                out_hbm.at[r_last, pl.ds(kk * _CHUNK, _CHUNK)],
                osems[kk % _NOBUF],
            ).wait()

    return k(v_flat, h)


def kernel(V, H):
    b, c, hd, wd = V.shape
    v_flat = V.reshape(b * c, hd * wd)
    out = _sc_gather(v_flat, H)
    return out.reshape(b, c, hd, wd)


# speedup vs baseline: 5.5158x; 1.3890x over previous
import functools

import jax
import jax.numpy as jnp
from jax import lax
from jax.experimental import pallas as pl
from jax.experimental.pallas import tpu as pltpu
from jax.experimental.pallas import tpu_sc as plsc

_B, _C, _HD, _WD = 8, 96, 224, 224
_HW = _HD * _WD
_NW = 32
_WPB = _NW // _B
_CPW = _C // _WPB
_NOBUF = 4
_CROWS = 16
_CHUNK = _CROWS * _WD
_NCHUNK = _HW // _CHUNK
_VPR = _WD // 16
_RSPLIT = 4
_RSUB = _HD // _RSPLIT
_ISPLIT = 4
_ISUB = _HW // _ISPLIT


def _sc_gather(v3, h):
    mesh = plsc.VectorSubcoreMesh(core_axis_name="c", subcore_axis_name="s")

    @functools.partial(
        pl.kernel,
        mesh=mesh,
        out_type=jax.ShapeDtypeStruct((_B * _C, _HD, _WD), jnp.float32),
        scratch_types=[
            pltpu.VMEM((_HW,), jnp.int32),
            pltpu.VMEM((_HD, _WD), jnp.float32),
            [pltpu.VMEM((_CROWS, _WD), jnp.float32)] * _NOBUF,
            [pltpu.SemaphoreType.DMA] * _NOBUF,
            pltpu.SemaphoreType.DMA,
        ],
        compiler_params=pltpu.CompilerParams(
            needs_layout_passes=False, use_tc_tiling_on_sc=True
        ),
    )
    def k(v_hbm, h_hbm, out_hbm, idx_v, row_v, outs, osems, rsem):
        cid = lax.axis_index("c")
        sid = lax.axis_index("s")
        wid = sid * 2 + cid
        b = wid // _WPB
        part = wid % _WPB

        icps = [
            pltpu.async_copy(
                h_hbm.at[b, pl.ds(t * _ISUB, _ISUB)],
                idx_v.at[pl.ds(t * _ISUB, _ISUB)],
                rsem,
            )
            for t in range(_ISPLIT)
        ]
        for cp in icps:
            cp.wait()

        def chan_body(j, carry):
            r = b * _C + part * _CPW + j
            rcps = [
                pltpu.async_copy(
                    v_hbm.at[r, pl.ds(t * _RSUB, _RSUB), :],
                    row_v.at[pl.ds(t * _RSUB, _RSUB), :],
                    rsem,
                )
                for t in range(_RSPLIT)
            ]
            for cp in rcps:
                cp.wait()

            ocps = [None] * _NCHUNK
            for kk in range(_NCHUNK):
                ov = outs[kk % _NOBUF]
                if kk >= _NOBUF:
                    ocps[kk - _NOBUF].wait()
                else:
                    @pl.when(j > 0)
                    def _():
                        pltpu.make_async_copy(
                            ov,
                            out_hbm.at[r, pl.ds(kk * _CROWS, _CROWS), :],
                            osems[kk % _NOBUF],
                        ).wait()

                @plsc.parallel_loop(0, _CHUNK // 16, unroll=8)
                def _(i):
                    src = idx_v[pl.ds(kk * _CHUNK + i * 16, 16)]
                    q = lax.shift_right_logical(src, 8)
                    m = lax.bitwise_and(src, 255)
                    vals = plsc.load_gather(row_v, [q, m])
                    orow = lax.div(i, _VPR)
                    ocol = lax.rem(i, _VPR) * 16
                    ov[orow, pl.ds(ocol, 16)] = vals

                ocps[kk] = pltpu.async_copy(
                    ov,
                    out_hbm.at[r, pl.ds(kk * _CROWS, _CROWS), :],
                    osems[kk % _NOBUF],
                )
            return carry

        lax.fori_loop(0, _CPW, chan_body, 0)

        r_last = b * _C + part * _CPW + (_CPW - 1)
        for kk in range(_NCHUNK - _NOBUF, _NCHUNK):
            pltpu.make_async_copy(
                outs[kk % _NOBUF],
                out_hbm.at[r_last, pl.ds(kk * _CROWS, _CROWS), :],
                osems[kk % _NOBUF],
            ).wait()

    return k(v3, h)


def kernel(V, H):
    b, c, hd, wd = V.shape
    v3 = V.reshape(b * c, hd, wd)
    hp = jnp.left_shift(H // wd, 8) | (H % wd)
    out = _sc_gather(v3, hp)
    return out.reshape(b, c, hd, wd)


# speedup vs baseline: 6.6969x; 1.2141x over previous
import functools

import jax
import jax.numpy as jnp
from jax import lax
from jax.experimental import pallas as pl
from jax.experimental.pallas import tpu as pltpu
from jax.experimental.pallas import tpu_sc as plsc

_B, _C, _HD, _WD = 8, 96, 224, 224
_HW = _HD * _WD
_NW = 32
_WPB = _NW // _B
_CPW = _C // _WPB
_CROWS = 8
_CHUNK = _CROWS * _WD
_NCHUNK = _HD // _CROWS
_VPR = _WD // 16
_RSPLIT = 4
_RSUB = _HD // _RSPLIT
_ISPLIT = 4
_ISUB = _HW // _ISPLIT


def _sc_gather(v3, hp):
    mesh = plsc.VectorSubcoreMesh(core_axis_name="c", subcore_axis_name="s")

    @functools.partial(
        pl.kernel,
        mesh=mesh,
        out_type=jax.ShapeDtypeStruct((_B * _C, _HD, _WD), jnp.float32),
        scratch_types=[
            pltpu.VMEM((_HW,), jnp.int32),
            pltpu.VMEM((_HD, _WD), jnp.float32),
            [pltpu.VMEM((_CROWS, _WD), jnp.float32)] * 2,
            [pltpu.SemaphoreType.DMA] * 2,
            pltpu.SemaphoreType.DMA,
        ],
        compiler_params=pltpu.CompilerParams(
            needs_layout_passes=False, use_tc_tiling_on_sc=True
        ),
    )
    def k(v_hbm, h_hbm, out_hbm, idx_v, row_v, outs, osems, rsem):
        cid = lax.axis_index("c")
        sid = lax.axis_index("s")
        wid = sid * 2 + cid
        b = wid // _WPB
        part = wid % _WPB

        icps = [
            pltpu.async_copy(
                h_hbm.at[b, pl.ds(t * _ISUB, _ISUB)],
                idx_v.at[pl.ds(t * _ISUB, _ISUB)],
                rsem,
            )
            for t in range(_ISPLIT)
        ]
        for cp in icps:
            cp.wait()

        def gather_chunk(base, ov):
            vecs = [(orow, vcol) for orow in range(_CROWS) for vcol in range(_VPR)]
            for w0 in range(0, len(vecs), 8):
                wave = vecs[w0 : w0 + 8]
                srcs = [
                    idx_v[pl.ds(base + orow * _WD + vcol * 16, 16)]
                    for (orow, vcol) in wave
                ]
                qs = [lax.shift_right_logical(s, 8) for s in srcs]
                ms = [lax.bitwise_and(s, 255) for s in srcs]
                vals = [
                    plsc.load_gather(row_v, [q, m]) for q, m in zip(qs, ms)
                ]
                for (orow, vcol), v in zip(wave, vals):
                    ov[orow, pl.ds(vcol * 16, 16)] = v

        def chan_body(j, carry):
            r = b * _C + part * _CPW + j
            rcps = [
                pltpu.async_copy(
                    v_hbm.at[r, pl.ds(t * _RSUB, _RSUB), :],
                    row_v.at[pl.ds(t * _RSUB, _RSUB), :],
                    rsem,
                )
                for t in range(_RSPLIT)
            ]
            for cp in rcps:
                cp.wait()

            def chunk_body(t, carry2):
                first = jnp.logical_and(j == 0, t == 0)
                for u in range(2):
                    kk = t * 2 + u

                    @pl.when(jnp.logical_not(first))
                    def _():
                        pltpu.make_async_copy(
                            outs[u],
                            out_hbm.at[r, pl.ds(kk * _CROWS, _CROWS), :],
                            osems[u],
                        ).wait()

                    gather_chunk(kk * _CHUNK, outs[u])
                    pltpu.async_copy(
                        outs[u],
                        out_hbm.at[r, pl.ds(kk * _CROWS, _CROWS), :],
                        osems[u],
                    )
                return carry2

            lax.fori_loop(0, _NCHUNK // 2, chunk_body, 0)
            return carry

        lax.fori_loop(0, _CPW, chan_body, 0)

        r_last = b * _C + part * _CPW + (_CPW - 1)
        for u in range(2):
            kk = _NCHUNK - 2 + u
            pltpu.make_async_copy(
                outs[u],
                out_hbm.at[r_last, pl.ds(kk * _CROWS, _CROWS), :],
                osems[u],
            ).wait()

    return k(v3, hp)


def kernel(V, H):
    b, c, hd, wd = V.shape
    v3 = V.reshape(b * c, hd, wd)
    hp = jnp.left_shift(H // wd, 8) | (H % wd)
    out = _sc_gather(v3, hp)
    return out.reshape(b, c, hd, wd)
